# baseline (speedup 1.0000x reference)
import functools

import numpy as np
import jax
import jax.numpy as jnp
from jax import lax
from jax.experimental import pallas as pl
from jax.experimental.pallas import tpu as pltpu
from jax.experimental.pallas import tpu_sc as plsc

NUM_EMBEDDINGS = 1000000
EMBEDDING_DIM = 64
NUM_BUCKETS = 100000
NUM_HASHES = 2
SEED = 42


def _is_prime(x):
    limit = int(np.sqrt(x))
    d = 2
    chunk = 1000000
    while d < limit:
        end = min(d + chunk, limit)
        ds = np.arange(d, end, dtype=np.int64)
        if np.any(x % ds == 0):
            return False
        d = end
    return True


def _next_prime(n):
    while not _is_prime(n):
        n += 1
    return n


def _hash_params():
    np.random.seed(SEED)
    moduler = _next_prime(int(np.random.randint(NUM_BUCKETS + 1, int(1000000000000000.0))))
    a_list, b_list = [], []
    sampled_a, sampled_b = set(), set()
    for _ in range(NUM_HASHES):
        np.random.seed(SEED)
        a = None
        while a is None or a in sampled_a:
            a = int(np.random.randint(1, moduler - 1))
        sampled_a.add(a)
        np.random.seed(SEED)
        b = None
        while b is None or b in sampled_b:
            b = int(np.random.randint(0, moduler - 1))
        sampled_b.add(b)
        a_list.append(a)
        b_list.append(b)
    return moduler, a_list, b_list


def _build_hash_tables():
    moduler, a_list, b_list = _hash_params()
    x = np.arange(NUM_EMBEDDINGS, dtype=np.int64)
    tabs = []
    for i in range(NUM_HASHES):
        prod = np.int64(a_list[i]) * x + np.int64(b_list[i])
        tabs.append(((prod % np.int64(moduler)) % np.int64(NUM_BUCKETS)).astype(np.int32))
    return tabs


_H0TAB, _H1TAB = _build_hash_tables()

_NC, _NS, _L = 2, 16, 16
_NW = _NC * _NS

_B, _F = 16384, 26
_N = _B * _F
_PW = _N // _NW
_BR = 128
_NB = _PW // _BR
_DOUT = EMBEDDING_DIM + NUM_HASHES


@functools.cache
def _make_embed_kernel():
    @functools.partial(
        pl.kernel,
        mesh=plsc.VectorSubcoreMesh(core_axis_name="c", subcore_axis_name="s"),
        out_type=(
            jax.ShapeDtypeStruct((_N // 2, 128), jnp.float32),
            jax.ShapeDtypeStruct((_N,), jnp.float32),
            jax.ShapeDtypeStruct((_N,), jnp.float32),
        ),
        compiler_params=pltpu.CompilerParams(
            needs_layout_passes=False, use_tc_tiling_on_sc=False),
        scratch_types=(
            [pltpu.VMEM((_BR,), jnp.int32)] * 2
            + [pltpu.VMEM((_BR,), jnp.int32)] * 2
            + [pltpu.VMEM((_BR,), jnp.float32)] * 2
            + [pltpu.VMEM((_BR,), jnp.int32)] * 2
            + [pltpu.VMEM((_BR,), jnp.float32)] * 2
            + [pltpu.VMEM((_BR, EMBEDDING_DIM), jnp.float32)] * 4
            + [pltpu.VMEM((_BR // 2, 128), jnp.float32)] * 2
            + [pltpu.VMEM((_BR,), jnp.float32)] * 4
            + [pltpu.SemaphoreType.DMA] * 6
        ),
    )
    def _embed_kernel(idx_hbm, h0t_hbm, h1t_hbm, emb_hbm, w0t_hbm, w1t_hbm,
                      out_hbm, w0out_hbm, w1out_hbm,
                      idxA, idxB, h0A, h1A, w0A, w1A, h0B, h1B, w0B, w1B,
                      e0A, e1A, e0B, e1B, outA, outB, wo0A, wo1A, wo0B, wo1B,
                      sem_i, sem_m, sem_eA, sem_eB, sem_oA, sem_oB):
        wid = lax.axis_index("s") * _NC + lax.axis_index("c")
        lane = lax.iota(jnp.int32, _L)
        bufA = (idxA, h0A, h1A, w0A, w1A, e0A, e1A, outA, wo0A, wo1A)
        bufB = (idxB, h0B, h1B, w0B, w1B, e0B, e1B, outB, wo0B, wo1B)

        def idx_slice(blk):
            return idx_hbm.at[pl.ds(wid * jnp.int32(_PW) + blk * jnp.int32(_BR), _BR)]

        def out_slices(blk):
            base = wid * jnp.int32(_PW) + blk * jnp.int32(_BR)
            return (
                out_hbm.at[pl.ds(wid * jnp.int32(_PW // 2) + blk * jnp.int32(_BR // 2),
                                 _BR // 2)],
                w0out_hbm.at[pl.ds(base, _BR)],
                w1out_hbm.at[pl.ds(base, _BR)],
            )

        def issue_idx(blk, buf):
            pltpu.async_copy(idx_slice(blk), buf[0], sem_i)

        def wait_idx(buf):
            pltpu.make_async_copy(idx_slice(jnp.int32(0)), buf[0], sem_i).wait()

        def issue_meta(buf):
            pltpu.async_copy(h0t_hbm.at[buf[0]], buf[1], sem_m)
            pltpu.async_copy(h1t_hbm.at[buf[0]], buf[2], sem_m)
            pltpu.async_copy(w0t_hbm.at[buf[0]], buf[3], sem_m)
            pltpu.async_copy(w1t_hbm.at[buf[0]], buf[4], sem_m)

        def wait_meta(buf):
            pltpu.make_async_copy(h0t_hbm.at[buf[0]], buf[1], sem_m).wait()
            pltpu.make_async_copy(h1t_hbm.at[buf[0]], buf[2], sem_m).wait()
            pltpu.make_async_copy(w0t_hbm.at[buf[0]], buf[3], sem_m).wait()
            pltpu.make_async_copy(w1t_hbm.at[buf[0]], buf[4], sem_m).wait()

        def issue_e(buf, sem):
            pltpu.async_copy(emb_hbm.at[buf[1]], buf[5], sem)
            pltpu.async_copy(emb_hbm.at[buf[2]], buf[6], sem)

        def wait_e(buf, sem):
            pltpu.make_async_copy(emb_hbm.at[buf[1]], buf[5], sem).wait()
            pltpu.make_async_copy(emb_hbm.at[buf[2]], buf[6], sem).wait()

        def issue_out(blk, buf, sem):
            om, ow0, ow1 = out_slices(blk)
            pltpu.async_copy(buf[7], om, sem)
            pltpu.async_copy(buf[8], ow0, sem)
            pltpu.async_copy(buf[9], ow1, sem)

        def wait_out(sem):
            om, ow0, ow1 = out_slices(jnp.int32(0))
            pltpu.make_async_copy(outA, om, sem).wait()
            pltpu.make_async_copy(wo0A, ow0, sem).wait()
            pltpu.make_async_copy(wo1A, ow1, sem).wait()

        def compute(buf):
            _, _, _, w0_v, w1_v, e0_v, e1_v, out_v, wo0_v, wo1_v = buf

            def row4(g, rc):
                for u in range(4):
                    r = jnp.int32(4) * g + jnp.int32(u)
                    w0 = plsc.load_gather(w0_v, [jnp.full((_L,), r, jnp.int32)])
                    w1 = plsc.load_gather(w1_v, [jnp.full((_L,), r, jnp.int32)])
                    ro = jnp.int32(2) * g + jnp.int32(u // 2)
                    co = 64 * (u % 2)
                    for c in range(EMBEDDING_DIM // _L):
                        out_v[ro, pl.ds(co + _L * c, _L)] = (
                            w0 * e0_v[r, pl.ds(_L * c, _L)]
                            + w1 * e1_v[r, pl.ds(_L * c, _L)])
                return rc

            lax.fori_loop(jnp.int32(0), jnp.int32(_BR // 4), row4, 0)

            def wstage(k, kc):
                s = pl.ds(_L * k, _L)
                wo0_v[s] = w0_v[s]
                wo1_v[s] = w1_v[s]
                return kc

            lax.fori_loop(jnp.int32(0), jnp.int32(_BR // _L), wstage, 0)

        def process(b, cur, nxt, sem_e_cur, sem_e_nxt, sem_o_cur):
            @pl.when(b < jnp.int32(_NB - 1))
            def _():
                wait_meta(nxt)
                issue_e(nxt, sem_e_nxt)

            @pl.when(b < jnp.int32(_NB - 2))
            def _():
                issue_idx(b + jnp.int32(2), cur)

            wait_e(cur, sem_e_cur)
            compute(cur)

            @pl.when(b >= jnp.int32(2))
            def _():
                wait_out(sem_o_cur)

            issue_out(b, cur, sem_o_cur)

            @pl.when(b < jnp.int32(_NB - 2))
            def _():
                wait_idx(cur)
                issue_meta(cur)

        issue_idx(jnp.int32(0), bufA)
        wait_idx(bufA)
        issue_meta(bufA)
        wait_meta(bufA)
        issue_e(bufA, sem_eA)
        issue_idx(jnp.int32(1), bufB)
        wait_idx(bufB)
        issue_meta(bufB)

        def pair(jj, carry):
            b0 = jnp.int32(2) * jj
            process(b0, bufA, bufB, sem_eA, sem_eB, sem_oA)
            process(b0 + jnp.int32(1), bufB, bufA, sem_eB, sem_eA, sem_oB)
            return carry

        lax.fori_loop(jnp.int32(0), jnp.int32(_NB // 2), pair, 0)

        wait_out(sem_oA)
        wait_out(sem_oB)

    return _embed_kernel


def kernel(indices, shared_embeddings, importance_weights):
    idx32 = (indices % NUM_EMBEDDINGS).astype(jnp.int32).reshape(_N)
    main, w0o, w1o = _make_embed_kernel()(
        idx32,
        jnp.asarray(_H0TAB),
        jnp.asarray(_H1TAB),
        shared_embeddings,
        importance_weights[:, 0],
        importance_weights[:, 1],
    )
    out = jnp.concatenate(
        [main.reshape(_N, EMBEDDING_DIM), w0o[:, None], w1o[:, None]], axis=1)
    return out.reshape(_B, _F, _DOUT)


# speedup vs baseline: 1.2904x; 1.2904x over previous
import functools

import numpy as np
import jax
import jax.numpy as jnp
from jax import lax
from jax.experimental import pallas as pl
from jax.experimental.pallas import tpu as pltpu
from jax.experimental.pallas import tpu_sc as plsc

NUM_EMBEDDINGS = 1000000
EMBEDDING_DIM = 64
NUM_BUCKETS = 100000
NUM_HASHES = 2
SEED = 42


def _is_prime(x):
    limit = int(np.sqrt(x))
    d = 2
    chunk = 1000000
    while d < limit:
        end = min(d + chunk, limit)
        ds = np.arange(d, end, dtype=np.int64)
        if np.any(x % ds == 0):
            return False
        d = end
    return True


def _next_prime(n):
    while not _is_prime(n):
        n += 1
    return n


def _hash_params():
    np.random.seed(SEED)
    moduler = _next_prime(int(np.random.randint(NUM_BUCKETS + 1, int(1000000000000000.0))))
    a_list, b_list = [], []
    sampled_a, sampled_b = set(), set()
    for _ in range(NUM_HASHES):
        np.random.seed(SEED)
        a = None
        while a is None or a in sampled_a:
            a = int(np.random.randint(1, moduler - 1))
        sampled_a.add(a)
        np.random.seed(SEED)
        b = None
        while b is None or b in sampled_b:
            b = int(np.random.randint(0, moduler - 1))
        sampled_b.add(b)
        a_list.append(a)
        b_list.append(b)
    return moduler, a_list, b_list


def _build_hash_tables():
    moduler, a_list, b_list = _hash_params()
    x = np.arange(NUM_EMBEDDINGS, dtype=np.int64)
    tabs = []
    for i in range(NUM_HASHES):
        prod = np.int64(a_list[i]) * x + np.int64(b_list[i])
        tabs.append(((prod % np.int64(moduler)) % np.int64(NUM_BUCKETS)).astype(np.int32))
    return tabs


_H0TAB, _H1TAB = _build_hash_tables()

_NC, _NS, _L = 2, 16, 16
_NW = _NC * _NS

_B, _F = 16384, 26
_N = _B * _F
_PW = _N // _NW
_BR = 128
_NB = _PW // _BR
_DOUT = EMBEDDING_DIM + NUM_HASHES


@functools.cache
def _make_embed_kernel():
    @functools.partial(
        pl.kernel,
        mesh=plsc.VectorSubcoreMesh(core_axis_name="c", subcore_axis_name="s"),
        out_type=(
            jax.ShapeDtypeStruct((_N // 2, 128), jnp.float32),
            jax.ShapeDtypeStruct((_N,), jnp.float32),
            jax.ShapeDtypeStruct((_N,), jnp.float32),
        ),
        compiler_params=pltpu.CompilerParams(
            needs_layout_passes=False, use_tc_tiling_on_sc=True),
        scratch_types=(
            [pltpu.VMEM((_BR,), jnp.int32)] * 2
            + [pltpu.VMEM((_BR,), jnp.int32)] * 2
            + [pltpu.VMEM((_BR,), jnp.float32)] * 2
            + [pltpu.VMEM((_BR,), jnp.int32)] * 2
            + [pltpu.VMEM((_BR,), jnp.float32)] * 2
            + [pltpu.VMEM((_BR, 128), jnp.float32)] * 4
            + [pltpu.VMEM((_BR // 2, 128), jnp.float32)] * 2
            + [pltpu.VMEM((_BR,), jnp.float32)] * 4
            + [pltpu.SemaphoreType.DMA] * 6
        ),
    )
    def _embed_kernel(idx_hbm, h0t_hbm, h1t_hbm, emb_hbm, w0t_hbm, w1t_hbm,
                      out_hbm, w0out_hbm, w1out_hbm,
                      idxA, idxB, h0A, h1A, w0A, w1A, h0B, h1B, w0B, w1B,
                      e0A, e1A, e0B, e1B, outA, outB, wo0A, wo1A, wo0B, wo1B,
                      sem_i, sem_m, sem_eA, sem_eB, sem_oA, sem_oB):
        wid = lax.axis_index("s") * _NC + lax.axis_index("c")
        lane = lax.iota(jnp.int32, _L)
        bufA = (idxA, h0A, h1A, w0A, w1A, e0A, e1A, outA, wo0A, wo1A)
        bufB = (idxB, h0B, h1B, w0B, w1B, e0B, e1B, outB, wo0B, wo1B)

        def idx_slice(blk):
            return idx_hbm.at[pl.ds(wid * jnp.int32(_PW) + blk * jnp.int32(_BR), _BR)]

        def out_slices(blk):
            base = wid * jnp.int32(_PW) + blk * jnp.int32(_BR)
            return (
                out_hbm.at[pl.ds(wid * jnp.int32(_PW // 2) + blk * jnp.int32(_BR // 2),
                                 _BR // 2)],
                w0out_hbm.at[pl.ds(base, _BR)],
                w1out_hbm.at[pl.ds(base, _BR)],
            )

        def issue_idx(blk, buf):
            pltpu.async_copy(idx_slice(blk), buf[0], sem_i)

        def wait_idx(buf):
            pltpu.make_async_copy(idx_slice(jnp.int32(0)), buf[0], sem_i).wait()

        def issue_meta(buf):
            pltpu.async_copy(h0t_hbm.at[buf[0]], buf[1], sem_m)
            pltpu.async_copy(h1t_hbm.at[buf[0]], buf[2], sem_m)
            pltpu.async_copy(w0t_hbm.at[buf[0]], buf[3], sem_m)
            pltpu.async_copy(w1t_hbm.at[buf[0]], buf[4], sem_m)

        def wait_meta(buf):
            pltpu.make_async_copy(h0t_hbm.at[buf[0]], buf[1], sem_m).wait()
            pltpu.make_async_copy(h1t_hbm.at[buf[0]], buf[2], sem_m).wait()
            pltpu.make_async_copy(w0t_hbm.at[buf[0]], buf[3], sem_m).wait()
            pltpu.make_async_copy(w1t_hbm.at[buf[0]], buf[4], sem_m).wait()

        def issue_e(buf, sem):
            pltpu.async_copy(emb_hbm.at[buf[1]], buf[5], sem)
            pltpu.async_copy(emb_hbm.at[buf[2]], buf[6], sem)

        def wait_e(buf, sem):
            pltpu.make_async_copy(emb_hbm.at[buf[1]], buf[5], sem).wait()
            pltpu.make_async_copy(emb_hbm.at[buf[2]], buf[6], sem).wait()

        def issue_out(blk, buf, sem):
            om, ow0, ow1 = out_slices(blk)
            pltpu.async_copy(buf[7], om, sem)
            pltpu.async_copy(buf[8], ow0, sem)
            pltpu.async_copy(buf[9], ow1, sem)

        def wait_out(sem):
            om, ow0, ow1 = out_slices(jnp.int32(0))
            pltpu.make_async_copy(outA, om, sem).wait()
            pltpu.make_async_copy(wo0A, ow0, sem).wait()
            pltpu.make_async_copy(wo1A, ow1, sem).wait()

        def compute(buf):
            _, _, _, w0_v, w1_v, e0_v, e1_v, out_v, wo0_v, wo1_v = buf

            @plsc.parallel_loop(
                jnp.int32(0), jnp.int32(_BR // 4), jnp.int32(1), unroll=2)
            def row4(g):
                for u in range(4):
                    r = jnp.int32(4) * g + jnp.int32(u)
                    w0 = plsc.load_gather(w0_v, [jnp.full((_L,), r, jnp.int32)])
                    w1 = plsc.load_gather(w1_v, [jnp.full((_L,), r, jnp.int32)])
                    ro = jnp.int32(2) * g + jnp.int32(u // 2)
                    co = 64 * (u % 2)
                    for c in range(EMBEDDING_DIM // _L):
                        out_v[ro, pl.ds(co + _L * c, _L)] = (
                            w0 * e0_v[r, pl.ds(_L * c, _L)]
                            + w1 * e1_v[r, pl.ds(_L * c, _L)])

            @plsc.parallel_loop(
                jnp.int32(0), jnp.int32(_BR // _L), jnp.int32(1), unroll=2)
            def wstage(k):
                s = pl.ds(_L * k, _L)
                wo0_v[s] = w0_v[s]
                wo1_v[s] = w1_v[s]

        def process(b, cur, nxt, sem_e_cur, sem_e_nxt, sem_o_cur):
            @pl.when(b < jnp.int32(_NB - 1))
            def _():
                wait_meta(nxt)
                issue_e(nxt, sem_e_nxt)

            @pl.when(b < jnp.int32(_NB - 2))
            def _():
                issue_idx(b + jnp.int32(2), cur)

            wait_e(cur, sem_e_cur)
            compute(cur)

            @pl.when(b >= jnp.int32(2))
            def _():
                wait_out(sem_o_cur)

            issue_out(b, cur, sem_o_cur)

            @pl.when(b < jnp.int32(_NB - 2))
            def _():
                wait_idx(cur)
                issue_meta(cur)

        issue_idx(jnp.int32(0), bufA)
        wait_idx(bufA)
        issue_meta(bufA)
        wait_meta(bufA)
        issue_e(bufA, sem_eA)
        issue_idx(jnp.int32(1), bufB)
        wait_idx(bufB)
        issue_meta(bufB)

        def pair(jj, carry):
            b0 = jnp.int32(2) * jj
            process(b0, bufA, bufB, sem_eA, sem_eB, sem_oA)
            process(b0 + jnp.int32(1), bufB, bufA, sem_eB, sem_eA, sem_oB)
            return carry

        lax.fori_loop(jnp.int32(0), jnp.int32(_NB // 2), pair, 0)

        wait_out(sem_oA)
        wait_out(sem_oB)

    return _embed_kernel


def kernel(indices, shared_embeddings, importance_weights):
    idx32 = (indices % NUM_EMBEDDINGS).astype(jnp.int32).reshape(_N)
    emb128 = jnp.pad(shared_embeddings, ((0, 0), (0, 128 - EMBEDDING_DIM)))
    main, w0o, w1o = _make_embed_kernel()(
        idx32,
        jnp.asarray(_H0TAB),
        jnp.asarray(_H1TAB),
        emb128,
        importance_weights[:, 0],
        importance_weights[:, 1],
    )
    out = jnp.concatenate(
        [main.reshape(_N, EMBEDDING_DIM), w0o[:, None], w1o[:, None]], axis=1)
    return out.reshape(_B, _F, _DOUT)


# speedup vs baseline: 1.6003x; 1.2402x over previous
import functools

import numpy as np
import jax
import jax.numpy as jnp
from jax import lax
from jax.experimental import pallas as pl
from jax.experimental.pallas import tpu as pltpu
from jax.experimental.pallas import tpu_sc as plsc

NUM_EMBEDDINGS = 1000000
EMBEDDING_DIM = 64
NUM_BUCKETS = 100000
NUM_HASHES = 2
SEED = 42


def _is_prime(x):
    limit = int(np.sqrt(x))
    d = 2
    chunk = 1000000
    while d < limit:
        end = min(d + chunk, limit)
        ds = np.arange(d, end, dtype=np.int64)
        if np.any(x % ds == 0):
            return False
        d = end
    return True


def _next_prime(n):
    while not _is_prime(n):
        n += 1
    return n


def _hash_params():
    np.random.seed(SEED)
    moduler = _next_prime(int(np.random.randint(NUM_BUCKETS + 1, int(1000000000000000.0))))
    a_list, b_list = [], []
    sampled_a, sampled_b = set(), set()
    for _ in range(NUM_HASHES):
        np.random.seed(SEED)
        a = None
        while a is None or a in sampled_a:
            a = int(np.random.randint(1, moduler - 1))
        sampled_a.add(a)
        np.random.seed(SEED)
        b = None
        while b is None or b in sampled_b:
            b = int(np.random.randint(0, moduler - 1))
        sampled_b.add(b)
        a_list.append(a)
        b_list.append(b)
    return moduler, a_list, b_list


def _build_hash_tables():
    moduler, a_list, b_list = _hash_params()
    x = np.arange(NUM_EMBEDDINGS, dtype=np.int64)
    tabs = []
    for i in range(NUM_HASHES):
        prod = np.int64(a_list[i]) * x + np.int64(b_list[i])
        tabs.append(((prod % np.int64(moduler)) % np.int64(NUM_BUCKETS)).astype(np.int32))
    return tabs


_H0TAB, _H1TAB = _build_hash_tables()

_NC, _NS, _L = 2, 16, 16
_NW = _NC * _NS

_B, _F = 16384, 26
_N = _B * _F
_PW = _N // _NW
_BR = 128
_NB = _PW // _BR
_DOUT = EMBEDDING_DIM + NUM_HASHES


@functools.cache
def _make_embed_kernel():
    @functools.partial(
        pl.kernel,
        mesh=plsc.VectorSubcoreMesh(core_axis_name="c", subcore_axis_name="s"),
        out_type=(
            jax.ShapeDtypeStruct((_N // 2, 128), jnp.float32),
            jax.ShapeDtypeStruct((_N,), jnp.float32),
            jax.ShapeDtypeStruct((_N,), jnp.float32),
        ),
        compiler_params=pltpu.CompilerParams(
            needs_layout_passes=False, use_tc_tiling_on_sc=True),
        scratch_types=(
            [pltpu.VMEM((_BR,), jnp.int32)] * 2
            + [pltpu.VMEM((_BR,), jnp.int32)] * 2
            + [pltpu.VMEM((_BR,), jnp.float32)] * 2
            + [pltpu.VMEM((_BR,), jnp.int32)] * 2
            + [pltpu.VMEM((_BR,), jnp.float32)] * 2
            + [pltpu.VMEM((_BR, 128), jnp.float32)] * 4
            + [pltpu.VMEM((_BR // 2, 128), jnp.float32)] * 2
            + [pltpu.VMEM((_BR,), jnp.float32)] * 4
            + [pltpu.SemaphoreType.DMA] * 6
        ),
    )
    def _embed_kernel(idx_hbm, h0t_hbm, h1t_hbm, emb_hbm, w0t_hbm, w1t_hbm,
                      out_hbm, w0out_hbm, w1out_hbm,
                      idxA, idxB, h0A, h1A, w0A, w1A, h0B, h1B, w0B, w1B,
                      e0A, e1A, e0B, e1B, outA, outB, wo0A, wo1A, wo0B, wo1B,
                      sem_i, sem_m, sem_eA, sem_eB, sem_oA, sem_oB):
        wid = lax.axis_index("s") * _NC + lax.axis_index("c")
        lane = lax.iota(jnp.int32, _L)
        bufA = (idxA, h0A, h1A, w0A, w1A, e0A, e1A, outA, wo0A, wo1A)
        bufB = (idxB, h0B, h1B, w0B, w1B, e0B, e1B, outB, wo0B, wo1B)

        def idx_slice(blk):
            return idx_hbm.at[pl.ds(wid * jnp.int32(_PW) + blk * jnp.int32(_BR), _BR)]

        def out_slices(blk):
            base = wid * jnp.int32(_PW) + blk * jnp.int32(_BR)
            return (
                out_hbm.at[pl.ds(wid * jnp.int32(_PW // 2) + blk * jnp.int32(_BR // 2),
                                 _BR // 2)],
                w0out_hbm.at[pl.ds(base, _BR)],
                w1out_hbm.at[pl.ds(base, _BR)],
            )

        def issue_idx(blk, buf):
            pltpu.async_copy(idx_slice(blk), buf[0], sem_i)

        def wait_idx(buf):
            pltpu.make_async_copy(idx_slice(jnp.int32(0)), buf[0], sem_i).wait()

        def issue_meta(buf):
            pltpu.async_copy(h0t_hbm.at[buf[0]], buf[1], sem_m)
            pltpu.async_copy(h1t_hbm.at[buf[0]], buf[2], sem_m)
            pltpu.async_copy(w0t_hbm.at[buf[0]], buf[3], sem_m)
            pltpu.async_copy(w1t_hbm.at[buf[0]], buf[4], sem_m)

        def wait_meta(buf):
            pltpu.make_async_copy(h0t_hbm.at[buf[0]], buf[1], sem_m).wait()
            pltpu.make_async_copy(h1t_hbm.at[buf[0]], buf[2], sem_m).wait()
            pltpu.make_async_copy(w0t_hbm.at[buf[0]], buf[3], sem_m).wait()
            pltpu.make_async_copy(w1t_hbm.at[buf[0]], buf[4], sem_m).wait()

        def issue_e(buf, sem):
            pltpu.async_copy(emb_hbm.at[buf[1]], buf[5], sem)
            pltpu.async_copy(emb_hbm.at[buf[2]], buf[6], sem)

        def wait_e(buf, sem):
            pltpu.make_async_copy(emb_hbm.at[buf[1]], buf[5], sem).wait()
            pltpu.make_async_copy(emb_hbm.at[buf[2]], buf[6], sem).wait()

        def issue_out(blk, buf, sem):
            om, ow0, ow1 = out_slices(blk)
            pltpu.async_copy(buf[7], om, sem)
            pltpu.async_copy(buf[8], ow0, sem)
            pltpu.async_copy(buf[9], ow1, sem)

        def wait_out(sem):
            om, ow0, ow1 = out_slices(jnp.int32(0))
            pltpu.make_async_copy(outA, om, sem).wait()
            pltpu.make_async_copy(wo0A, ow0, sem).wait()
            pltpu.make_async_copy(wo1A, ow1, sem).wait()

        def compute(buf):
            _, _, _, w0_v, w1_v, e0_v, e1_v, out_v, wo0_v, wo1_v = buf

            @plsc.parallel_loop(
                jnp.int32(0), jnp.int32(_BR // 4), jnp.int32(1), unroll=2)
            def row4(g):
                for u in range(4):
                    r = jnp.int32(4) * g + jnp.int32(u)
                    w0 = plsc.load_gather(w0_v, [jnp.full((_L,), r, jnp.int32)])
                    w1 = plsc.load_gather(w1_v, [jnp.full((_L,), r, jnp.int32)])
                    ro = jnp.int32(2) * g + jnp.int32(u // 2)
                    co = 64 * (u % 2)
                    for c in range(EMBEDDING_DIM // _L):
                        out_v[ro, pl.ds(co + _L * c, _L)] = (
                            w0 * e0_v[r, pl.ds(_L * c, _L)]
                            + w1 * e1_v[r, pl.ds(_L * c, _L)])

            @plsc.parallel_loop(
                jnp.int32(0), jnp.int32(_BR // _L), jnp.int32(1), unroll=2)
            def wstage(k):
                s = pl.ds(_L * k, _L)
                wo0_v[s] = w0_v[s]
                wo1_v[s] = w1_v[s]

        def process(b, cur, nxt, sem_e_cur, sem_e_nxt, sem_o_cur):
            @pl.when(b < jnp.int32(_NB - 1))
            def _():
                wait_meta(nxt)
                issue_e(nxt, sem_e_nxt)

            @pl.when(b < jnp.int32(_NB - 2))
            def _():
                issue_idx(b + jnp.int32(2), cur)

            wait_e(cur, sem_e_cur)
            compute(cur)

            @pl.when(b >= jnp.int32(2))
            def _():
                wait_out(sem_o_cur)

            issue_out(b, cur, sem_o_cur)

            @pl.when(b < jnp.int32(_NB - 2))
            def _():
                wait_idx(cur)
                issue_meta(cur)

        issue_idx(jnp.int32(0), bufA)
        wait_idx(bufA)
        issue_meta(bufA)
        wait_meta(bufA)
        issue_e(bufA, sem_eA)
        issue_idx(jnp.int32(1), bufB)
        wait_idx(bufB)
        issue_meta(bufB)

        def pair(jj, carry):
            b0 = jnp.int32(2) * jj
            process(b0, bufA, bufB, sem_eA, sem_eB, sem_oA)
            process(b0 + jnp.int32(1), bufB, bufA, sem_eB, sem_eA, sem_oB)
            return carry

        lax.fori_loop(jnp.int32(0), jnp.int32(_NB // 2), pair, 0)

        wait_out(sem_oA)
        wait_out(sem_oB)

    return _embed_kernel


def kernel(indices, shared_embeddings, importance_weights):
    idx32 = (indices.astype(jnp.int32) % jnp.int32(NUM_EMBEDDINGS)).reshape(_N)
    emb128 = jnp.pad(shared_embeddings, ((0, 0), (0, 128 - EMBEDDING_DIM)))
    main, w0o, w1o = _make_embed_kernel()(
        idx32,
        jnp.asarray(_H0TAB),
        jnp.asarray(_H1TAB),
        emb128,
        importance_weights[:, 0],
        importance_weights[:, 1],
    )
    out = jnp.concatenate(
        [main.reshape(_N, EMBEDDING_DIM), w0o[:, None], w1o[:, None]], axis=1)
    return out.reshape(_B, _F, _DOUT)
